# Initial kernel scaffold; baseline (speedup 1.0000x reference)
import jax
import jax.numpy as jnp
from jax.experimental import pallas as pl


def kernel(x, tpl_edge_index, W1, b1, g1, be1, W2, b2, g2, be2, W3, b3, g3, be3):
    raise NotImplementedError("write your pallas kernel here")


# speedup vs baseline: 1.0312x; 1.0312x over previous
import functools

import jax
import jax.numpy as jnp
import numpy as np
from jax.experimental import pallas as pl

BN_EPS = 1e-5
D = 128


def _final_mlp_body(a_ref, w_ref, b_ref, s_ref, be_ref, o_ref):
    z = jnp.dot(a_ref[...], w_ref[...], preferred_element_type=jnp.float32)
    z = z + b_ref[...]
    h = jnp.maximum(z, 0.0)
    o_ref[...] = h * s_ref[...] + be_ref[...]


def _final_mlp(agg, W3, b3, s3, be3):
    n = agg.shape[0]
    blk = 2000
    grid = n // blk
    return pl.pallas_call(
        _final_mlp_body,
        grid=(grid,),
        in_specs=[
            pl.BlockSpec((blk, D), lambda i: (i, 0)),
            pl.BlockSpec((D, D), lambda i: (0, 0)),
            pl.BlockSpec((1, D), lambda i: (0, 0)),
            pl.BlockSpec((1, D), lambda i: (0, 0)),
            pl.BlockSpec((1, D), lambda i: (0, 0)),
        ],
        out_specs=pl.BlockSpec((blk, D), lambda i: (i, 0)),
        out_shape=jax.ShapeDtypeStruct((n, D), jnp.float32),
    )(agg, W3.T, b3.reshape(1, D), s3.reshape(1, D), be3.reshape(1, D))


def kernel(x, tpl_edge_index, W1, b1, g1, be1, W2, b2, g2, be2, W3, b3, g3, be3):
    n = x.shape[0]
    inv = 1.0 / np.sqrt(1.0 + BN_EPS)
    s1 = g1 * inv
    s2 = g2 * inv
    s3 = g3 * inv

    src = tpl_edge_index[0]
    dst = tpl_edge_index[1]
    loop = jnp.arange(n, dtype=src.dtype)
    src = jnp.concatenate([src, loop])
    dst = jnp.concatenate([dst, loop])

    W1a = W1[:, :D]
    W1b = W1[:, D:]
    U = x @ (W1a - W1b).T
    V = x @ W1b.T

    z1 = U[dst] + V[src] + b1
    h = jnp.maximum(z1, 0.0) * s1 + be1
    z2 = h @ W2.T + b2
    m2 = jnp.maximum(z2, 0.0) * s2 + be2
    agg = jax.ops.segment_max(m2, dst, num_segments=n)

    return _final_mlp(agg, W3, b3, s3, be3)


# speedup vs baseline: 2.3526x; 2.2814x over previous
import functools

import jax
import jax.numpy as jnp
import numpy as np
from jax.experimental import pallas as pl
from jax.experimental.pallas import tpu as pltpu
from jax.experimental.pallas import tpu_sc as plsc

BN_EPS = 1e-5
D = 128
N_NODES = 10000
GATHER_WIN = 128
NW = 32


def _uv_body(x_ref, wd_ref, wb_ref, u_ref, v_ref):
    xb = x_ref[...]
    u_ref[...] = jnp.dot(xb, wd_ref[...], preferred_element_type=jnp.float32)
    v_ref[...] = jnp.dot(xb, wb_ref[...], preferred_element_type=jnp.float32)


def _uv_project(x, WdT, WbT):
    n = x.shape[0]
    blk = 2000
    return pl.pallas_call(
        _uv_body,
        grid=(n // blk,),
        in_specs=[
            pl.BlockSpec((blk, D), lambda i: (i, 0)),
            pl.BlockSpec((D, D), lambda i: (0, 0)),
            pl.BlockSpec((D, D), lambda i: (0, 0)),
        ],
        out_specs=[
            pl.BlockSpec((blk, D), lambda i: (i, 0)),
            pl.BlockSpec((blk, D), lambda i: (i, 0)),
        ],
        out_shape=[
            jax.ShapeDtypeStruct((n, D), jnp.float32),
            jax.ShapeDtypeStruct((n, D), jnp.float32),
        ],
    )(x, WdT, WbT)


def _sc_gather(U, V, dst_e, src_e, e_pad):
    mesh = plsc.VectorSubcoreMesh(core_axis_name="c", subcore_axis_name="s")

    @functools.partial(
        pl.kernel,
        out_type=[
            jax.ShapeDtypeStruct((e_pad, D), jnp.float32),
            jax.ShapeDtypeStruct((e_pad, D), jnp.float32),
        ],
        mesh=mesh,
    )
    def gather_kernel(u_hbm, v_hbm, di_hbm, si_hbm, gu_hbm, gv_hbm):
        def body(di_v, si_v, gu_v, gv_v):
            pltpu.sync_copy(u_hbm.at[di_v.at[0]], gu_v)
            pltpu.sync_copy(v_hbm.at[si_v.at[0]], gv_v)

        pltpu.emit_pipeline(
            body,
            grid=(e_pad // GATHER_WIN,),
            in_specs=[
                pl.BlockSpec((1, GATHER_WIN), index_map=lambda i: (0, i)),
                pl.BlockSpec((1, GATHER_WIN), index_map=lambda i: (0, i)),
            ],
            out_specs=[
                pl.BlockSpec((GATHER_WIN, D), index_map=lambda i: (i, 0)),
                pl.BlockSpec((GATHER_WIN, D), index_map=lambda i: (i, 0)),
            ],
            core_axis_name=("c", "s"),
            dimension_semantics=(pltpu.PARALLEL,),
        )(di_hbm, si_hbm, gu_hbm, gv_hbm)

    return gather_kernel(U, V, dst_e.reshape(1, e_pad), src_e.reshape(1, e_pad))


def _edge_mlp_body(gu_ref, gv_ref, b1_ref, w2_ref, b2_ref, s2_ref, be2_ref,
                   m2_ref):
    z1 = gu_ref[...] + gv_ref[...] + b1_ref[...]
    h = jnp.maximum(z1, 0.0)
    z2 = jnp.dot(h, w2_ref[...], preferred_element_type=jnp.float32)
    z2 = z2 + b2_ref[...]
    m2_ref[...] = jnp.maximum(z2, 0.0) * s2_ref[...] + be2_ref[...]


def _edge_mlp(gU, gV, b1, W2p, b2p, s2, be2):
    e_pad = gU.shape[0]
    blk = 2048
    row = lambda a: a.reshape(1, D)
    return pl.pallas_call(
        _edge_mlp_body,
        grid=(e_pad // blk,),
        in_specs=[
            pl.BlockSpec((blk, D), lambda i: (i, 0)),
            pl.BlockSpec((blk, D), lambda i: (i, 0)),
            pl.BlockSpec((1, D), lambda i: (0, 0)),
            pl.BlockSpec((D, D), lambda i: (0, 0)),
            pl.BlockSpec((1, D), lambda i: (0, 0)),
            pl.BlockSpec((1, D), lambda i: (0, 0)),
            pl.BlockSpec((1, D), lambda i: (0, 0)),
        ],
        out_specs=pl.BlockSpec((blk, D), lambda i: (i, 0)),
        out_shape=jax.ShapeDtypeStruct((e_pad, D), jnp.float32),
    )(gU, gV, row(b1), W2p, row(b2p), row(s2), row(be2))


def _final_mlp_body(a_ref, w_ref, b_ref, s_ref, be_ref, o_ref):
    z = jnp.dot(a_ref[...], w_ref[...], preferred_element_type=jnp.float32)
    z = z + b_ref[...]
    h = jnp.maximum(z, 0.0)
    o_ref[...] = h * s_ref[...] + be_ref[...]


def _final_mlp(agg, W3T, b3, s3, be3):
    n = agg.shape[0]
    blk = 2000
    row = lambda a: a.reshape(1, D)
    return pl.pallas_call(
        _final_mlp_body,
        grid=(n // blk,),
        in_specs=[
            pl.BlockSpec((blk, D), lambda i: (i, 0)),
            pl.BlockSpec((D, D), lambda i: (0, 0)),
            pl.BlockSpec((1, D), lambda i: (0, 0)),
            pl.BlockSpec((1, D), lambda i: (0, 0)),
            pl.BlockSpec((1, D), lambda i: (0, 0)),
        ],
        out_specs=pl.BlockSpec((blk, D), lambda i: (i, 0)),
        out_shape=jax.ShapeDtypeStruct((n, D), jnp.float32),
    )(agg, W3T, row(b3), row(s3), row(be3))


def kernel(x, tpl_edge_index, W1, b1, g1, be1, W2, b2, g2, be2, W3, b3, g3, be3):
    n = x.shape[0]
    inv = 1.0 / np.sqrt(1.0 + BN_EPS)
    s1 = g1 * inv
    s2 = g2 * inv
    s3 = g3 * inv
    W2p = (W2 * s1[None, :]).T
    b2p = b2 + W2 @ be1

    src = tpl_edge_index[0].astype(jnp.int32)
    dst = tpl_edge_index[1].astype(jnp.int32)
    n_edges = src.shape[0]
    loop = jnp.arange(n, dtype=jnp.int32)
    e_real = n_edges + n
    e_pad = ((e_real + GATHER_WIN * NW - 1) // (GATHER_WIN * NW)) * (GATHER_WIN * NW)
    pad = e_pad - e_real
    src_e = jnp.concatenate([src, loop, jnp.zeros((pad,), jnp.int32)])
    dst_e = jnp.concatenate([dst, loop, jnp.zeros((pad,), jnp.int32)])

    W1a = W1[:, :D]
    W1b = W1[:, D:]
    U, V = _uv_project(x, (W1a - W1b).T, W1b.T)

    gU, gV = _sc_gather(U, V, dst_e, src_e, e_pad)
    m2 = _edge_mlp(gU, gV, b1, W2p, b2p, s2, be2)

    agg = jax.ops.segment_max(m2, dst_e, num_segments=n)

    return _final_mlp(agg, W3.T, b3, s3, be3)
